# baseline (speedup 1.0000x reference)
import functools

import jax
import jax.numpy as jnp
from jax import lax
from jax.experimental import pallas as pl
from jax.experimental.pallas import tpu as pltpu
from jax.experimental.pallas import tpu_sc as plsc

_VOCAB = 1000000
_D = 32
_B = 4096
_L = 200
_N = _B * _L
_NW = 32
_UNIT = 1024
_NUNITS = _N // _UNIT
_UPW = _NUNITS // _NW
_NTILES = _N * _D // (8 * 128)

_mesh = plsc.VectorSubcoreMesh(core_axis_name="c", subcore_axis_name="s")


@functools.partial(
    pl.kernel,
    out_type=jax.ShapeDtypeStruct((_NTILES, 8, 128), jnp.float32),
    mesh=_mesh,
    scratch_types=[
        pltpu.VMEM((2, _UNIT), jnp.int32),
        pltpu.VMEM((2, _UNIT, _D), jnp.float32),
        pltpu.VMEM((8, 128), jnp.float32),
        pltpu.VMEM((8, 128), jnp.float32),
        pltpu.SemaphoreType.DMA,
        pltpu.SemaphoreType.DMA,
        pltpu.SemaphoreType.DMA,
        pltpu.SemaphoreType.DMA,
    ],
    compiler_params=pltpu.CompilerParams(use_tc_tiling_on_sc=False,
                                         needs_layout_passes=False),
)
def _emb_lookup(idx_hbm, table_hbm, out_hbm, ibuf, gbuf, t0, t1,
                sidx, sg, st0, st1):
    wid = lax.axis_index("s") * 2 + lax.axis_index("c")
    u0 = wid * _UPW
    iota = lax.iota(jnp.int32, 16)
    tsems = (st0, st1)
    tbufs = (t0, t1)

    def idx_copy(u, par):
        return pltpu.make_async_copy(
            idx_hbm.at[pl.ds((u0 + u) * _UNIT, _UNIT)], ibuf.at[par], sidx)

    def gather_copy(par):
        return pltpu.make_async_copy(table_hbm.at[ibuf.at[par]],
                                     gbuf.at[par], sg)

    idx_copy(0, 0).start()
    idx_copy(0, 0).wait()
    gather_copy(0).start()

    def unit_body(u, carry):
        par = lax.rem(u, 2)
        nxt = lax.rem(u + 1, 2)

        @pl.when(u + 1 < _UPW)
        def _prefetch():
            idx_copy(u + 1, nxt).start()
            idx_copy(u + 1, nxt).wait()

        gather_copy(par).wait()

        @pl.when(u + 1 < _UPW)
        def _next_gather():
            gather_copy(nxt).start()

        g = u0 + u
        lt = g // 32
        bt = lax.rem(g, 32)
        parv = jnp.full((16,), par, jnp.int32)

        def pair_body(t, carry2):
            for half, (tb, ts) in enumerate(zip(tbufs, tsems)):
                tt = 2 * t + half
                i = tt // 4
                dblk = lax.rem(tt, 4)

                @pl.when(jnp.logical_or(u > 0, t > 0))
                def _wait_prev_tile():
                    pltpu.make_async_copy(tb, out_hbm.at[0], ts).wait()

                for di in range(8):
                    colv = jnp.full((16,), 8 * dblk + di, jnp.int32)
                    for jb in range(8):
                        rowv = jnp.full((16,), i * 128 + jb * 16,
                                        jnp.int32) + iota
                        vals = plsc.load_gather(gbuf, [parv, rowv, colv])
                        tb[di, pl.ds(jb * 16, 16)] = vals
                tidx = ((8 * lt + i) * 4 + dblk) * 32 + bt
                pltpu.make_async_copy(tb, out_hbm.at[tidx], ts).start()
            return carry2

        lax.fori_loop(0, 16, pair_body, 0)
        return carry

    lax.fori_loop(0, _UPW, unit_body, 0)
    pltpu.make_async_copy(t0, out_hbm.at[0], st0).wait()
    pltpu.make_async_copy(t1, out_hbm.at[0], st1).wait()


def kernel(input_ids, weight):
    ids_perm = (input_ids.T.reshape(_L // 8, 8, _B // 128, 128)
                .transpose(0, 2, 1, 3).reshape(-1).astype(jnp.int32))
    out3 = _emb_lookup(ids_perm, weight)
    out = (out3.reshape(_L, _D // 8, _B // 128, 8, 128)
           .transpose(2, 4, 0, 1, 3).reshape(_B, _L, _D))
    return out
